# baseline (speedup 1.0000x reference)
import functools

import jax
import jax.numpy as jnp
import numpy as np
from jax import lax
from jax.experimental import pallas as pl
from jax.experimental.pallas import tpu as pltpu
from jax.experimental.pallas import tpu_sc as plsc

N = 10000
E = 320000
D = 128

NC = 2
NS = 16
ROWS_PER_TILE = 640
N_PAD = NS * ROWS_PER_TILE

DEG_LANES = 128
DEG_CH = 160
DEG_PAD = NS * DEG_CH * DEG_LANES - E

E_HALF = E // NC
CHUNK = 64
SP_PHASES = 2
PH_CH = 80
SP_CH = SP_PHASES * PH_CH
SP_PAD = NS * SP_CH * CHUNK - E_HALF
NBUF = 4

_mesh = plsc.VectorSubcoreMesh(core_axis_name="c", subcore_axis_name="s")


def _deg_body(idx_hbm, zeros_hbm, deg_hbm, idx_v, ones_v, deg_sh):
    c = lax.axis_index("c")
    s = lax.axis_index("s")
    pltpu.sync_copy(idx_hbm.at[c, s], idx_v)
    for i in range(8):
        ones_v[pl.ds(16 * i, 16)] = jnp.ones((16,), jnp.float32)
    pltpu.sync_copy(zeros_hbm.at[pl.ds(s * ROWS_PER_TILE, ROWS_PER_TILE)],
                    deg_sh.at[pl.ds(s * ROWS_PER_TILE, ROWS_PER_TILE)])
    plsc.subcore_barrier()

    def body(j, carry):
        pltpu.sync_copy(ones_v, deg_sh.at[idx_v.at[j]], add=True)
        return carry

    lax.fori_loop(0, DEG_CH, body, 0)
    plsc.subcore_barrier()
    pltpu.sync_copy(deg_sh.at[pl.ds(s * ROWS_PER_TILE, ROWS_PER_TILE)],
                    deg_hbm.at[c, pl.ds(s * ROWS_PER_TILE, ROWS_PER_TILE)])


_deg_kernel = functools.partial(
    pl.kernel,
    out_type=jax.ShapeDtypeStruct((NC, N_PAD), jnp.float32),
    mesh=_mesh,
    scratch_types=[
        pltpu.VMEM((DEG_CH, DEG_LANES), jnp.int32),
        pltpu.VMEM((DEG_LANES,), jnp.float32),
        pltpu.VMEM_SHARED((N_PAD,), jnp.float32),
    ],
)(_deg_body)


def _spmm_body(ht_hbm, src_hbm, dst_hbm, zeros_hbm, out_hbm,
               src_v, dst_v, rows, agg_sh, sems):
    c = lax.axis_index("c")
    s = lax.axis_index("s")
    pltpu.sync_copy(zeros_hbm.at[pl.ds(s * ROWS_PER_TILE, ROWS_PER_TILE), :],
                    agg_sh.at[pl.ds(s * ROWS_PER_TILE, ROWS_PER_TILE), :])
    plsc.subcore_barrier()

    def src_slice(row, half):
        return src_v.at[row, pl.ds(CHUNK * half, CHUNK)]

    for p in range(SP_PHASES):
        pltpu.sync_copy(src_hbm.at[c, s, p], src_v)
        pltpu.sync_copy(dst_hbm.at[c, s, p], dst_v)

        for b in range(NBUF):
            pltpu.async_copy(ht_hbm.at[src_slice(b // 2, b % 2)],
                             rows[b], sems[b])

        def body(g, carry):
            j0 = NBUF * g
            for b in range(NBUF):
                pltpu.make_async_copy(
                    ht_hbm.at[src_slice(j0 // 2 + b // 2, b % 2)],
                    rows[b], sems[b]).wait()
                pltpu.sync_copy(rows[b], agg_sh.at[dst_v.at[j0 + b]],
                                add=True)
                pltpu.async_copy(
                    ht_hbm.at[src_slice(j0 // 2 + 2 + b // 2, b % 2)],
                    rows[b], sems[b])
            return carry

        lax.fori_loop(0, PH_CH // NBUF - 1, body, 0)
        for b in range(NBUF):
            j = PH_CH - NBUF + b
            pltpu.make_async_copy(
                ht_hbm.at[src_slice(j // 2, b % 2)],
                rows[b], sems[b]).wait()
            pltpu.sync_copy(rows[b], agg_sh.at[dst_v.at[j]], add=True)

    plsc.subcore_barrier()
    pltpu.sync_copy(agg_sh.at[pl.ds(s * ROWS_PER_TILE, ROWS_PER_TILE), :],
                    out_hbm.at[c, pl.ds(s * ROWS_PER_TILE, ROWS_PER_TILE), :])


def _spmm_entry(ht_hbm, src_hbm, dst_hbm, zeros_hbm, out_hbm, src_v, dst_v,
                r0, r1, r2, r3, agg_sh, sem0, sem1, sem2, sem3):
    _spmm_body(ht_hbm, src_hbm, dst_hbm, zeros_hbm, out_hbm, src_v, dst_v,
               (r0, r1, r2, r3), agg_sh, (sem0, sem1, sem2, sem3))


_spmm_kernel = functools.partial(
    pl.kernel,
    out_type=jax.ShapeDtypeStruct((NC, N_PAD, D), jnp.float32),
    mesh=_mesh,
    scratch_types=[
        pltpu.VMEM((PH_CH // 2, 2 * CHUNK), jnp.int32),
        pltpu.VMEM((PH_CH, CHUNK), jnp.int32),
        pltpu.VMEM((CHUNK, D), jnp.float32),
        pltpu.VMEM((CHUNK, D), jnp.float32),
        pltpu.VMEM((CHUNK, D), jnp.float32),
        pltpu.VMEM((CHUNK, D), jnp.float32),
        pltpu.VMEM_SHARED((N_PAD, D), jnp.float32),
        pltpu.SemaphoreType.DMA,
        pltpu.SemaphoreType.DMA,
        pltpu.SemaphoreType.DMA,
        pltpu.SemaphoreType.DMA,
    ],
)(_spmm_entry)


BLK = 1000
NBLK = N // BLK


def _pre_body(x_ref, w_ref, degs_ref, ht_ref, dinv_ref):
    dinv = lax.rsqrt(jnp.clip(degs_ref[...], 1.0, None))
    dinv_ref[...] = dinv
    ht_ref[...] = jnp.dot(x_ref[...] * dinv[:, 0:1], w_ref[...],
                          preferred_element_type=jnp.float32)


def _pre_call(x, w, degs):
    return pl.pallas_call(
        _pre_body,
        grid=(NBLK,),
        in_specs=[
            pl.BlockSpec((BLK, D), lambda i: (i, 0)),
            pl.BlockSpec((D, D), lambda i: (0, 0)),
            pl.BlockSpec((BLK, 2), lambda i: (i, 0)),
        ],
        out_specs=(
            pl.BlockSpec((BLK, D), lambda i: (i, 0)),
            pl.BlockSpec((BLK, 2), lambda i: (i, 0)),
        ),
        out_shape=(
            jax.ShapeDtypeStruct((N, D), jnp.float32),
            jax.ShapeDtypeStruct((N, 2), jnp.float32),
        ),
    )(x, w, degs)


def _post_body(p_ref, dinv_ref, b_ref, g_ref, bt_ref, w_ref, out_ref,
               h_scr, stats):
    ph = pl.program_id(0)
    i = pl.program_id(1)
    dinv = dinv_ref[...]

    @pl.when(ph == 0)
    def _phase0():
        h = (p_ref[0] + p_ref[1]) * dinv[:, 1:2] + b_ref[...][None, :]
        h = jnp.maximum(h, 0.0)
        h_scr[pl.ds(i * BLK, BLK), :] = h

        @pl.when(i == 0)
        def _init():
            stats[...] = jnp.zeros((2, D), jnp.float32)

        stats[0:1, :] += jnp.sum(h, axis=0, keepdims=True)
        stats[1:2, :] += jnp.sum(h * h, axis=0, keepdims=True)

    @pl.when(ph == 1)
    def _phase1():
        mu = stats[0:1, :] * (1.0 / N)
        var = stats[1:2, :] * (1.0 / N) - mu * mu
        h = h_scr[pl.ds(i * BLK, BLK), :]
        hn = (h - mu) * lax.rsqrt(var + 1e-5)
        hn = hn * g_ref[...][None, :] + bt_ref[...][None, :]
        out_ref[...] = jnp.dot(hn * dinv[:, 0:1], w_ref[...],
                               preferred_element_type=jnp.float32)


def _post_call(partials, dinv, b, g, bt, w):
    vec = pl.BlockSpec((D,), lambda ph, i: (0,))
    return pl.pallas_call(
        _post_body,
        grid=(2, NBLK),
        in_specs=[
            pl.BlockSpec((2, BLK, D), lambda ph, i: (0, i * (1 - ph), 0)),
            pl.BlockSpec((BLK, 2), lambda ph, i: (i, 0)),
            vec, vec, vec,
            pl.BlockSpec((D, D), lambda ph, i: (0, 0)),
        ],
        out_specs=pl.BlockSpec((BLK, D), lambda ph, i: (i, 0)),
        out_shape=jax.ShapeDtypeStruct((N, D), jnp.float32),
        scratch_shapes=[
            pltpu.VMEM((N, D), jnp.float32),
            pltpu.VMEM((2, D), jnp.float32),
        ],
    )(partials, dinv, b, g, bt, w)


def _final_body(p_ref, dinv_ref, b_ref, out_ref):
    p = p_ref[0] + p_ref[1]
    h = p * dinv_ref[...][:, 1:2] + b_ref[...][None, :]
    m = jnp.max(h, axis=1, keepdims=True)
    e = h - m
    lse = jnp.log(jnp.sum(jnp.exp(e), axis=1, keepdims=True))
    out_ref[...] = e - lse


def _final_call(partials, dinv, b):
    return pl.pallas_call(
        _final_body,
        grid=(NBLK,),
        in_specs=[
            pl.BlockSpec((2, BLK, D), lambda i: (0, i, 0)),
            pl.BlockSpec((BLK, 2), lambda i: (i, 0)),
            pl.BlockSpec((D,), lambda i: (0,)),
        ],
        out_specs=pl.BlockSpec((BLK, D), lambda i: (i, 0)),
        out_shape=jax.ShapeDtypeStruct((N, D), jnp.float32),
    )(partials, dinv, b)


_DEG_PAD_IDX = (N + np.arange(DEG_PAD) % (N_PAD - N)).astype(np.int32)
_SP_PAD_SRC = ((np.arange(SP_PAD) * 37) % N).astype(np.int32)
_SP_PAD_DST = (N + np.arange(SP_PAD) % (N_PAD - N)).astype(np.int32)


def kernel(x, edge_index, W1, b1, W2, b2, W3, b3, g1, bt1, g2, bt2):
    src = edge_index[0].astype(jnp.int32)
    dst = edge_index[1].astype(jnp.int32)

    deg_idx = jnp.stack([
        jnp.concatenate([src, _DEG_PAD_IDX]).reshape(NS, DEG_CH, DEG_LANES),
        jnp.concatenate([dst, _DEG_PAD_IDX]).reshape(NS, DEG_CH, DEG_LANES),
    ])
    src_shape = (NS, SP_PHASES, PH_CH // 2, 2 * CHUNK)
    dst_shape = (NS, SP_PHASES, PH_CH, CHUNK)
    sp_src = jnp.stack([
        jnp.concatenate([src[:E_HALF], _SP_PAD_SRC]).reshape(src_shape),
        jnp.concatenate([src[E_HALF:], _SP_PAD_SRC]).reshape(src_shape),
    ])
    sp_dst = jnp.stack([
        jnp.concatenate([dst[:E_HALF], _SP_PAD_DST]).reshape(dst_shape),
        jnp.concatenate([dst[E_HALF:], _SP_PAD_DST]).reshape(dst_shape),
    ])
    zeros1 = jnp.zeros((N_PAD,), jnp.float32)
    zeros2 = jnp.zeros((N_PAD, D), jnp.float32)

    deg = _deg_kernel(deg_idx, zeros1)
    degs = deg[:, :N].T

    ht1, dinv = _pre_call(x, W1, degs)
    p1 = _spmm_kernel(ht1, sp_src, sp_dst, zeros2)
    ht2 = _post_call(p1, dinv, b1, g1, bt1, W2)
    p2 = _spmm_kernel(ht2, sp_src, sp_dst, zeros2)
    ht3 = _post_call(p2, dinv, b2, g2, bt2, W3)
    p3 = _spmm_kernel(ht3, sp_src, sp_dst, zeros2)
    return _final_call(p3, dinv, b3)


# speedup vs baseline: 1.0581x; 1.0581x over previous
import functools

import jax
import jax.numpy as jnp
import numpy as np
from jax import lax
from jax.experimental import pallas as pl
from jax.experimental.pallas import tpu as pltpu
from jax.experimental.pallas import tpu_sc as plsc

N = 10000
E = 320000
D = 128

NC = 2
NS = 16
ROWS_PER_TILE = 640
N_PAD = NS * ROWS_PER_TILE

DEG_LANES = 128
DEG_CH = 160
DEG_PAD = NS * DEG_CH * DEG_LANES - E

E_HALF = E // NC
CHUNK = 64
SP_PHASES = 2
PH_CH = 80
SP_CH = SP_PHASES * PH_CH
SP_PAD = NS * SP_CH * CHUNK - E_HALF
NBUF = 4

_mesh = plsc.VectorSubcoreMesh(core_axis_name="c", subcore_axis_name="s")


def _deg_body(idx_hbm, zeros_hbm, deg_hbm, idx_v, ones_v, deg_sh):
    c = lax.axis_index("c")
    s = lax.axis_index("s")
    pltpu.sync_copy(idx_hbm.at[c, s], idx_v)
    for i in range(8):
        ones_v[pl.ds(16 * i, 16)] = jnp.ones((16,), jnp.float32)
    pltpu.sync_copy(zeros_hbm.at[pl.ds(s * ROWS_PER_TILE, ROWS_PER_TILE)],
                    deg_sh.at[pl.ds(s * ROWS_PER_TILE, ROWS_PER_TILE)])
    plsc.subcore_barrier()

    def body(j, carry):
        pltpu.sync_copy(ones_v, deg_sh.at[idx_v.at[j]], add=True)
        return carry

    lax.fori_loop(0, DEG_CH, body, 0)
    plsc.subcore_barrier()
    pltpu.sync_copy(deg_sh.at[pl.ds(s * ROWS_PER_TILE, ROWS_PER_TILE)],
                    deg_hbm.at[c, pl.ds(s * ROWS_PER_TILE, ROWS_PER_TILE)])


_deg_kernel = functools.partial(
    pl.kernel,
    out_type=jax.ShapeDtypeStruct((NC, N_PAD), jnp.float32),
    mesh=_mesh,
    scratch_types=[
        pltpu.VMEM((DEG_CH, DEG_LANES), jnp.int32),
        pltpu.VMEM((DEG_LANES,), jnp.float32),
        pltpu.VMEM_SHARED((N_PAD,), jnp.float32),
    ],
)(_deg_body)


def _spmm_body(ht_hbm, src_hbm, dst_hbm, zeros_hbm, out_hbm,
               src_v, dst_v, rows, agg_sh, sems):
    c = lax.axis_index("c")
    s = lax.axis_index("s")
    pltpu.sync_copy(zeros_hbm.at[pl.ds(s * ROWS_PER_TILE, ROWS_PER_TILE), :],
                    agg_sh.at[pl.ds(s * ROWS_PER_TILE, ROWS_PER_TILE), :])
    plsc.subcore_barrier()

    def src_slice(row, half):
        return src_v.at[row, pl.ds(CHUNK * half, CHUNK)]

    for p in range(SP_PHASES):
        pltpu.sync_copy(src_hbm.at[c, s, p], src_v)
        pltpu.sync_copy(dst_hbm.at[c, s, p], dst_v)

        for b in range(NBUF):
            pltpu.async_copy(ht_hbm.at[src_slice(b // 2, b % 2)],
                             rows[b], sems[b])

        def body(g, carry):
            j0 = NBUF * g
            for b in range(NBUF):
                pltpu.make_async_copy(
                    ht_hbm.at[src_slice(j0 // 2 + b // 2, b % 2)],
                    rows[b], sems[b]).wait()
                pltpu.sync_copy(rows[b], agg_sh.at[dst_v.at[j0 + b]],
                                add=True)
                pltpu.async_copy(
                    ht_hbm.at[src_slice(j0 // 2 + 2 + b // 2, b % 2)],
                    rows[b], sems[b])
            return carry

        lax.fori_loop(0, PH_CH // NBUF - 1, body, 0)
        for b in range(NBUF):
            j = PH_CH - NBUF + b
            pltpu.make_async_copy(
                ht_hbm.at[src_slice(j // 2, b % 2)],
                rows[b], sems[b]).wait()
            pltpu.sync_copy(rows[b], agg_sh.at[dst_v.at[j]], add=True)

    plsc.subcore_barrier()
    pltpu.sync_copy(agg_sh.at[pl.ds(s * ROWS_PER_TILE, ROWS_PER_TILE), :],
                    out_hbm.at[c, pl.ds(s * ROWS_PER_TILE, ROWS_PER_TILE), :])


def _spmm_entry(ht_hbm, src_hbm, dst_hbm, zeros_hbm, out_hbm, src_v, dst_v,
                r0, r1, r2, r3, agg_sh, sem0, sem1, sem2, sem3):
    _spmm_body(ht_hbm, src_hbm, dst_hbm, zeros_hbm, out_hbm, src_v, dst_v,
               (r0, r1, r2, r3), agg_sh, (sem0, sem1, sem2, sem3))


_spmm_kernel = functools.partial(
    pl.kernel,
    out_type=jax.ShapeDtypeStruct((NC, N_PAD, D), jnp.float32),
    mesh=_mesh,
    scratch_types=[
        pltpu.VMEM((PH_CH // 2, 2 * CHUNK), jnp.int32),
        pltpu.VMEM((PH_CH, CHUNK), jnp.int32),
        pltpu.VMEM((CHUNK, D), jnp.float32),
        pltpu.VMEM((CHUNK, D), jnp.float32),
        pltpu.VMEM((CHUNK, D), jnp.float32),
        pltpu.VMEM((CHUNK, D), jnp.float32),
        pltpu.VMEM_SHARED((N_PAD, D), jnp.float32),
        pltpu.SemaphoreType.DMA,
        pltpu.SemaphoreType.DMA,
        pltpu.SemaphoreType.DMA,
        pltpu.SemaphoreType.DMA,
    ],
)(_spmm_entry)


def _pre_body(x_ref, w_ref, degs_ref, ht_ref, dinv_ref):
    dinv = lax.rsqrt(jnp.clip(degs_ref[...], 1.0, None))
    dinv_ref[...] = dinv
    ht_ref[...] = jnp.dot(x_ref[...] * dinv[:, 0:1], w_ref[...],
                          preferred_element_type=jnp.float32)


def _pre_call(x, w, degs):
    return pl.pallas_call(
        _pre_body,
        out_shape=(
            jax.ShapeDtypeStruct((N, D), jnp.float32),
            jax.ShapeDtypeStruct((N, 2), jnp.float32),
        ),
    )(x, w, degs)


def _post_body(p_ref, dinv_ref, b_ref, g_ref, bt_ref, w_ref, out_ref):
    p = p_ref[0, :N, :] + p_ref[1, :N, :]
    dinv = dinv_ref[...]
    h = p * dinv[:, 1:2] + b_ref[...][None, :]
    h = jnp.maximum(h, 0.0)
    mu = jnp.mean(h, axis=0, keepdims=True)
    var = jnp.mean((h - mu) * (h - mu), axis=0, keepdims=True)
    hn = (h - mu) * lax.rsqrt(var + 1e-5)
    hn = hn * g_ref[...][None, :] + bt_ref[...][None, :]
    out_ref[...] = jnp.dot(hn * dinv[:, 0:1], w_ref[...],
                           preferred_element_type=jnp.float32)


def _post_call(partials, dinv, b, g, bt, w):
    return pl.pallas_call(
        _post_body,
        out_shape=jax.ShapeDtypeStruct((N, D), jnp.float32),
    )(partials, dinv, b, g, bt, w)


def _final_body(p_ref, dinv_ref, b_ref, out_ref):
    p = p_ref[0, :N, :] + p_ref[1, :N, :]
    h = p * dinv_ref[...][:, 1:2] + b_ref[...][None, :]
    m = jnp.max(h, axis=1, keepdims=True)
    e = h - m
    lse = jnp.log(jnp.sum(jnp.exp(e), axis=1, keepdims=True))
    out_ref[...] = e - lse


def _final_call(partials, dinv, b):
    return pl.pallas_call(
        _final_body,
        out_shape=jax.ShapeDtypeStruct((N, D), jnp.float32),
    )(partials, dinv, b)


_DEG_PAD_IDX = (N + np.arange(DEG_PAD) % (N_PAD - N)).astype(np.int32)
_SP_PAD_SRC = ((np.arange(SP_PAD) * 37) % N).astype(np.int32)
_SP_PAD_DST = (N + np.arange(SP_PAD) % (N_PAD - N)).astype(np.int32)


def kernel(x, edge_index, W1, b1, W2, b2, W3, b3, g1, bt1, g2, bt2):
    src = edge_index[0].astype(jnp.int32)
    dst = edge_index[1].astype(jnp.int32)

    deg_idx = jnp.stack([
        jnp.concatenate([src, _DEG_PAD_IDX]).reshape(NS, DEG_CH, DEG_LANES),
        jnp.concatenate([dst, _DEG_PAD_IDX]).reshape(NS, DEG_CH, DEG_LANES),
    ])
    src_shape = (NS, SP_PHASES, PH_CH // 2, 2 * CHUNK)
    dst_shape = (NS, SP_PHASES, PH_CH, CHUNK)
    sp_src = jnp.stack([
        jnp.concatenate([src[:E_HALF], _SP_PAD_SRC]).reshape(src_shape),
        jnp.concatenate([src[E_HALF:], _SP_PAD_SRC]).reshape(src_shape),
    ])
    sp_dst = jnp.stack([
        jnp.concatenate([dst[:E_HALF], _SP_PAD_DST]).reshape(dst_shape),
        jnp.concatenate([dst[E_HALF:], _SP_PAD_DST]).reshape(dst_shape),
    ])
    zeros1 = jnp.zeros((N_PAD,), jnp.float32)
    zeros2 = jnp.zeros((N_PAD, D), jnp.float32)

    deg = _deg_kernel(deg_idx, zeros1)
    degs = deg[:, :N].T

    ht1, dinv = _pre_call(x, W1, degs)
    p1 = _spmm_kernel(ht1, sp_src, sp_dst, zeros2)
    ht2 = _post_call(p1, dinv, b1, g1, bt1, W2)
    p2 = _spmm_kernel(ht2, sp_src, sp_dst, zeros2)
    ht3 = _post_call(p2, dinv, b2, g2, bt2, W3)
    p3 = _spmm_kernel(ht3, sp_src, sp_dst, zeros2)
    return _final_call(p3, dinv, b3)


# speedup vs baseline: 1.0731x; 1.0142x over previous
import functools

import jax
import jax.numpy as jnp
import numpy as np
from jax import lax
from jax.experimental import pallas as pl
from jax.experimental.pallas import tpu as pltpu
from jax.experimental.pallas import tpu_sc as plsc

N = 10000
E = 320000
D = 128

NC = 2
NS = 16
ROWS_PER_TILE = 640
N_PAD = NS * ROWS_PER_TILE

DEG_LANES = 128
DEG_CH = 160
DEG_PAD = NS * DEG_CH * DEG_LANES - E

E_HALF = E // NC
CHUNK = 64
SP_PHASES = 2
PH_CH = 80
SP_CH = SP_PHASES * PH_CH
SP_PAD = NS * SP_CH * CHUNK - E_HALF
NBUF = 4

_mesh = plsc.VectorSubcoreMesh(core_axis_name="c", subcore_axis_name="s")


def _deg_body(idx_hbm, zeros_hbm, deg_hbm, idx_v, ones_v, deg_sh):
    c = lax.axis_index("c")
    s = lax.axis_index("s")
    pltpu.sync_copy(idx_hbm.at[c, s], idx_v)
    for i in range(8):
        ones_v[pl.ds(16 * i, 16)] = jnp.ones((16,), jnp.float32)
    pltpu.sync_copy(zeros_hbm.at[pl.ds(s * ROWS_PER_TILE, ROWS_PER_TILE)],
                    deg_sh.at[pl.ds(s * ROWS_PER_TILE, ROWS_PER_TILE)])
    plsc.subcore_barrier()

    def body(j, carry):
        pltpu.sync_copy(ones_v, deg_sh.at[idx_v.at[j]], add=True)
        return carry

    lax.fori_loop(0, DEG_CH, body, 0)
    plsc.subcore_barrier()
    pltpu.sync_copy(deg_sh.at[pl.ds(s * ROWS_PER_TILE, ROWS_PER_TILE)],
                    deg_hbm.at[c, pl.ds(s * ROWS_PER_TILE, ROWS_PER_TILE)])


_deg_kernel = functools.partial(
    pl.kernel,
    out_type=jax.ShapeDtypeStruct((NC, N_PAD), jnp.float32),
    mesh=_mesh,
    scratch_types=[
        pltpu.VMEM((DEG_CH, DEG_LANES), jnp.int32),
        pltpu.VMEM((DEG_LANES,), jnp.float32),
        pltpu.VMEM_SHARED((N_PAD,), jnp.float32),
    ],
)(_deg_body)


def _spmm_body(ht_hbm, src_hbm, dst_hbm, zeros_hbm, out_hbm,
               src_v, dst_v, rows, agg_sh, sems):
    c = lax.axis_index("c")
    s = lax.axis_index("s")
    pltpu.sync_copy(zeros_hbm.at[pl.ds(s * ROWS_PER_TILE, ROWS_PER_TILE), :],
                    agg_sh.at[pl.ds(s * ROWS_PER_TILE, ROWS_PER_TILE), :])
    plsc.subcore_barrier()

    def src_slice(row, half):
        return src_v.at[row, pl.ds(CHUNK * half, CHUNK)]

    for p in range(SP_PHASES):
        pltpu.sync_copy(src_hbm.at[c, s, p], src_v)
        pltpu.sync_copy(dst_hbm.at[c, s, p], dst_v)

        for b in range(NBUF):
            pltpu.async_copy(ht_hbm.at[src_slice(b // 2, b % 2)],
                             rows[b], sems[b])

        def body(g, carry):
            j0 = NBUF * g
            for b in range(NBUF):
                pltpu.make_async_copy(
                    ht_hbm.at[src_slice(j0 // 2 + b // 2, b % 2)],
                    rows[b], sems[b]).wait()
                pltpu.sync_copy(rows[b], agg_sh.at[dst_v.at[j0 + b]],
                                add=True)
                pltpu.async_copy(
                    ht_hbm.at[src_slice(j0 // 2 + 2 + b // 2, b % 2)],
                    rows[b], sems[b])
            return carry

        lax.fori_loop(0, PH_CH // NBUF - 1, body, 0)
        for b in range(NBUF):
            j = PH_CH - NBUF + b
            pltpu.make_async_copy(
                ht_hbm.at[src_slice(j // 2, b % 2)],
                rows[b], sems[b]).wait()
            pltpu.sync_copy(rows[b], agg_sh.at[dst_v.at[j]], add=True)

    plsc.subcore_barrier()
    pltpu.sync_copy(agg_sh.at[pl.ds(s * ROWS_PER_TILE, ROWS_PER_TILE), :],
                    out_hbm.at[c, pl.ds(s * ROWS_PER_TILE, ROWS_PER_TILE), :])


def _spmm_entry(ht_hbm, src_hbm, dst_hbm, zeros_hbm, out_hbm, src_v, dst_v,
                r0, r1, r2, r3, agg_sh, sem0, sem1, sem2, sem3):
    _spmm_body(ht_hbm, src_hbm, dst_hbm, zeros_hbm, out_hbm, src_v, dst_v,
               (r0, r1, r2, r3), agg_sh, (sem0, sem1, sem2, sem3))


_spmm_kernel = functools.partial(
    pl.kernel,
    out_type=jax.ShapeDtypeStruct((NC, N_PAD, D), jnp.float32),
    mesh=_mesh,
    scratch_types=[
        pltpu.VMEM((PH_CH // 2, 2 * CHUNK), jnp.int32),
        pltpu.VMEM((PH_CH, CHUNK), jnp.int32),
        pltpu.VMEM((CHUNK, D), jnp.float32),
        pltpu.VMEM((CHUNK, D), jnp.float32),
        pltpu.VMEM((CHUNK, D), jnp.float32),
        pltpu.VMEM((CHUNK, D), jnp.float32),
        pltpu.VMEM_SHARED((N_PAD, D), jnp.float32),
        pltpu.SemaphoreType.DMA,
        pltpu.SemaphoreType.DMA,
        pltpu.SemaphoreType.DMA,
        pltpu.SemaphoreType.DMA,
    ],
)(_spmm_entry)


def _pre_body(x_ref, w_ref, deg_ref, ht_ref, dinv_ref):
    dinv2 = lax.rsqrt(jnp.clip(deg_ref[...], 1.0, None))
    dinv = jnp.transpose(dinv2[:, :N])
    dinv_ref[...] = dinv
    ht_ref[...] = jnp.dot(x_ref[...] * dinv[:, 0:1], w_ref[...],
                          preferred_element_type=jnp.float32)


def _pre_call(x, w, deg):
    return pl.pallas_call(
        _pre_body,
        out_shape=(
            jax.ShapeDtypeStruct((N, D), jnp.float32),
            jax.ShapeDtypeStruct((N, 2), jnp.float32),
        ),
    )(x, w, deg)


def _post_body(p_ref, dinv_ref, b_ref, g_ref, bt_ref, w_ref, out_ref):
    p = p_ref[0, :N, :] + p_ref[1, :N, :]
    dinv = dinv_ref[...]
    h = p * dinv[:, 1:2] + b_ref[...][None, :]
    h = jnp.maximum(h, 0.0)
    mu = jnp.mean(h, axis=0, keepdims=True)
    var = jnp.mean((h - mu) * (h - mu), axis=0, keepdims=True)
    hn = (h - mu) * lax.rsqrt(var + 1e-5)
    hn = hn * g_ref[...][None, :] + bt_ref[...][None, :]
    out_ref[...] = jnp.dot(hn * dinv[:, 0:1], w_ref[...],
                           preferred_element_type=jnp.float32)


def _post_call(partials, dinv, b, g, bt, w):
    return pl.pallas_call(
        _post_body,
        out_shape=jax.ShapeDtypeStruct((N, D), jnp.float32),
    )(partials, dinv, b, g, bt, w)


def _final_body(p_ref, dinv_ref, b_ref, out_ref):
    p = p_ref[0, :N, :] + p_ref[1, :N, :]
    h = p * dinv_ref[...][:, 1:2] + b_ref[...][None, :]
    m = jnp.max(h, axis=1, keepdims=True)
    e = h - m
    lse = jnp.log(jnp.sum(jnp.exp(e), axis=1, keepdims=True))
    out_ref[...] = e - lse


def _final_call(partials, dinv, b):
    return pl.pallas_call(
        _final_body,
        out_shape=jax.ShapeDtypeStruct((N, D), jnp.float32),
    )(partials, dinv, b)


_DEG_PAD_IDX = (N + np.arange(DEG_PAD) % (N_PAD - N)).astype(np.int32)
_SP_PAD_SRC = ((np.arange(SP_PAD) * 37) % N).astype(np.int32)
_SP_PAD_DST = (N + np.arange(SP_PAD) % (N_PAD - N)).astype(np.int32)


def kernel(x, edge_index, W1, b1, W2, b2, W3, b3, g1, bt1, g2, bt2):
    src = edge_index[0].astype(jnp.int32)
    dst = edge_index[1].astype(jnp.int32)

    deg_idx = jnp.stack([
        jnp.concatenate([src, _DEG_PAD_IDX]).reshape(NS, DEG_CH, DEG_LANES),
        jnp.concatenate([dst, _DEG_PAD_IDX]).reshape(NS, DEG_CH, DEG_LANES),
    ])
    src_shape = (NS, SP_PHASES, PH_CH // 2, 2 * CHUNK)
    dst_shape = (NS, SP_PHASES, PH_CH, CHUNK)
    sp_src = jnp.stack([
        jnp.concatenate([src[:E_HALF], _SP_PAD_SRC]).reshape(src_shape),
        jnp.concatenate([src[E_HALF:], _SP_PAD_SRC]).reshape(src_shape),
    ])
    sp_dst = jnp.stack([
        jnp.concatenate([dst[:E_HALF], _SP_PAD_DST]).reshape(dst_shape),
        jnp.concatenate([dst[E_HALF:], _SP_PAD_DST]).reshape(dst_shape),
    ])
    zeros1 = jnp.zeros((N_PAD,), jnp.float32)
    zeros2 = jnp.zeros((N_PAD, D), jnp.float32)

    deg = _deg_kernel(deg_idx, zeros1)

    ht1, dinv = _pre_call(x, W1, deg)
    p1 = _spmm_kernel(ht1, sp_src, sp_dst, zeros2)
    ht2 = _post_call(p1, dinv, b1, g1, bt1, W2)
    p2 = _spmm_kernel(ht2, sp_src, sp_dst, zeros2)
    ht3 = _post_call(p2, dinv, b2, g2, bt2, W3)
    p3 = _spmm_kernel(ht3, sp_src, sp_dst, zeros2)
    return _final_call(p3, dinv, b3)
